# baseline (speedup 1.0000x reference)
import functools

import jax
import jax.numpy as jnp
from jax import lax
from jax.experimental import pallas as pl
from jax.experimental.pallas import tpu as pltpu
from jax.experimental.pallas import tpu_sc as plsc

B = 1024
Q = 100
C = 92
K = 4
L = 16
CH = 46
BIG = 10**9

_mesh = plsc.VectorSubcoreMesh(core_axis_name="c", subcore_axis_name="s")


@functools.partial(
    pl.kernel,
    mesh=_mesh,
    compiler_params=pltpu.CompilerParams(needs_layout_passes=False),
    out_type=jax.ShapeDtypeStruct((Q, B), jnp.float32),
    scratch_types=[
        pltpu.VMEM((CH, 8, 128), jnp.float32),
        pltpu.VMEM((CH, 8, 128), jnp.float32),
        pltpu.VMEM((8, 128), jnp.float32),
        pltpu.VMEM((8, 128), jnp.float32),
        pltpu.VMEM((128,), jnp.float32),
        pltpu.VMEM((4, 128), jnp.float32),
        pltpu.VMEM((Q, 128), jnp.float32),
        pltpu.VMEM_SHARED((16, 128), jnp.float32),
        pltpu.SemaphoreType.DMA,
        pltpu.SemaphoreType.DMA,
    ],
)
def _mask_kernel(lt_hbm, out_hbm, bufa, bufb, mpart, mtile, accbuf, pbuf,
                 scorebuf, shacc, sema, semb):
    core = lax.axis_index("c")
    s = lax.axis_index("s")
    tj = core * 4 + s // 4
    k = s % 4
    bcol = tj * 128

    def qoff_of(u):
        return pl.multiple_of((k * 3 + u) * 8, 8)

    def half_src(u, h):
        return lt_hbm.at[pl.ds(h * CH, CH), pl.ds(qoff_of(u), 8),
                         pl.ds(bcol, 128)]

    def tail_src(h):
        return lt_hbm.at[pl.ds(h * CH, CH), pl.ds(96, 4), pl.ds(bcol, 128)]

    for g in range(8):
        accbuf[pl.ds(g * L, L)] = jnp.zeros((L,), jnp.float32)

    pltpu.async_copy(half_src(0, 0), bufa, sema)

    def unit(u, _):
        qoff = qoff_of(u)
        pltpu.make_async_copy(half_src(u, 0), bufa, sema).wait()
        pltpu.async_copy(half_src(u, 1), bufb, semb)

        def p1row(r, _):
            qq, g = r // 8, r % 8
            m = bufa[0, qq, pl.ds(g * L, L)]

            def cstep(i, mm):
                c0 = 1 + i * 9
                v = [bufa[c0 + d, qq, pl.ds(g * L, L)] for d in range(9)]
                t = jnp.maximum(jnp.maximum(v[0], v[1]),
                                jnp.maximum(v[2], v[3]))
                t2 = jnp.maximum(jnp.maximum(v[4], v[5]),
                                 jnp.maximum(v[6], v[7]))
                return jnp.maximum(mm, jnp.maximum(jnp.maximum(t, t2), v[8]))

            mpart[qq, pl.ds(g * L, L)] = lax.fori_loop(0, 5, cstep, m)
            return 0

        lax.fori_loop(0, 64, p1row, 0)
        pltpu.make_async_copy(half_src(u, 1), bufb, semb).wait()

        @pl.when(u < 2)
        def _():
            pltpu.async_copy(half_src(u + 1, 0), bufa, sema)

        def p2row(r, _):
            qq, g = r // 8, r % 8
            m = mpart[qq, pl.ds(g * L, L)]

            def cstep(i, mm):
                c0 = i * 9
                v = [bufb[c0 + d, qq, pl.ds(g * L, L)] for d in range(9)]
                t = jnp.maximum(jnp.maximum(v[0], v[1]),
                                jnp.maximum(v[2], v[3]))
                t2 = jnp.maximum(jnp.maximum(v[4], v[5]),
                                 jnp.maximum(v[6], v[7]))
                return jnp.maximum(mm, jnp.maximum(jnp.maximum(t, t2), v[8]))

            m = lax.fori_loop(0, 5, cstep, m)
            l91 = bufb[CH - 1, qq, pl.ds(g * L, L)]
            m1 = jnp.where(m >= l91, 1.0, 0.0)
            mtile[qq, pl.ds(g * L, L)] = m1
            accbuf[pl.ds(g * L, L)] = accbuf[pl.ds(g * L, L)] + m1
            return 0

        lax.fori_loop(0, 64, p2row, 0)
        pltpu.sync_copy(mtile,
                        out_hbm.at[pl.ds(qoff, 8), pl.ds(bcol, 128)])
        return 0

    lax.fori_loop(0, 3, unit, 0)

    @pl.when(k == 3)
    def _tail():
        pltpu.sync_copy(tail_src(0), bufa.at[:, pl.ds(0, 4), :])
        pltpu.sync_copy(tail_src(1), bufb.at[:, pl.ds(0, 4), :])

        def trow(r, _):
            qq, g = r // 8, r % 8
            m = bufa[0, qq, pl.ds(g * L, L)]

            def c1(i, mm):
                c0 = 1 + i * 9
                for d in range(9):
                    mm = jnp.maximum(mm, bufa[c0 + d, qq, pl.ds(g * L, L)])
                return mm

            m = lax.fori_loop(0, 5, c1, m)

            def c2(i, mm):
                c0 = i * 9
                for d in range(9):
                    mm = jnp.maximum(mm, bufb[c0 + d, qq, pl.ds(g * L, L)])
                return mm

            m = lax.fori_loop(0, 5, c2, m)
            l91 = bufb[CH - 1, qq, pl.ds(g * L, L)]
            m1 = jnp.where(m >= l91, 1.0, 0.0)
            mtile[qq, pl.ds(g * L, L)] = m1
            accbuf[pl.ds(g * L, L)] = accbuf[pl.ds(g * L, L)] + m1
            return 0

        lax.fori_loop(0, 32, trow, 0)
        pltpu.sync_copy(mtile.at[pl.ds(0, 4), :],
                        out_hbm.at[pl.ds(96, 4), pl.ds(bcol, 128)])

    pltpu.sync_copy(accbuf, shacc.at[s])
    plsc.subcore_barrier()
    pltpu.sync_copy(shacc.at[pl.ds((s // 4) * 4, 4)], pbuf)
    tot = [pbuf[0, pl.ds(g * L, L)] + pbuf[1, pl.ds(g * L, L)]
           + pbuf[2, pl.ds(g * L, L)] + pbuf[3, pl.ds(g * L, L)]
           for g in range(8)]
    zvec = sum(jnp.where(tot[g] < 1.0, 1.0, 0.0) for g in range(8))
    zany = jnp.max(zvec) > 0.0

    def fallback():
        def srows(bufx, bufy, qbase, nrow):
            def srow(r, _):
                qq, g = r // 8, r % 8
                ds = pl.ds(g * L, L)
                m = bufx[0, qq, ds]

                def c1(cc, mm):
                    return jnp.maximum(mm, bufx[cc, qq, ds])

                m = lax.fori_loop(1, CH, c1, m)

                def c2(cc, mm):
                    return jnp.maximum(mm, bufy[cc, qq, ds])

                m = lax.fori_loop(0, CH - 1, c2, m)

                def e1(cc, ss):
                    return ss + jnp.exp(bufx[cc, qq, ds] - m)

                ssum = lax.fori_loop(0, CH, e1,
                                     jnp.zeros((L,), jnp.float32))

                def e2(cc, ss):
                    return ss + jnp.exp(bufy[cc, qq, ds] - m)

                ssum = lax.fori_loop(0, CH - 1, e2, ssum)
                scorebuf[qbase + qq, ds] = 1.0 / ssum
                return 0

            lax.fori_loop(0, nrow * 8, srow, 0)

        def stile(ti, _):
            qoff = pl.multiple_of(ti * 8, 8)
            pltpu.sync_copy(lt_hbm.at[pl.ds(0, CH), pl.ds(qoff, 8),
                                      pl.ds(bcol, 128)], bufa)
            pltpu.sync_copy(lt_hbm.at[pl.ds(CH, CH), pl.ds(qoff, 8),
                                      pl.ds(bcol, 128)], bufb)
            srows(bufa, bufb, qoff, 8)
            return 0

        lax.fori_loop(0, 12, stile, 0)
        pltpu.sync_copy(tail_src(0), bufa.at[:, pl.ds(0, 4), :])
        pltpu.sync_copy(tail_src(1), bufb.at[:, pl.ds(0, 4), :])
        srows(bufa, bufb, 96, 4)

        def kround(_, __):
            for g in range(8):
                ds = pl.ds(g * L, L)
                mx = scorebuf[0, ds]

                def qmax(q, mm):
                    return jnp.maximum(mm, scorebuf[q, ds])

                mx = lax.fori_loop(1, Q, qmax, mx)

                def qidx(q, bi):
                    return jnp.minimum(
                        bi, jnp.where(scorebuf[q, ds] == mx, q, BIG))

                bidx = lax.fori_loop(0, Q, qidx,
                                     jnp.full((L,), BIG, jnp.int32))

                def qkill(q, _):
                    row = scorebuf[q, ds]
                    scorebuf[q, ds] = jnp.where(bidx == q, -2.0, row)
                    return 0

                lax.fori_loop(0, Q, qkill, 0)
            return 0

        lax.fori_loop(0, K, kround, 0)

        def fix_rows(qbase, nrow, src):
            pltpu.sync_copy(src, mtile.at[pl.ds(0, nrow), :])
            for g in range(8):
                ds = pl.ds(g * L, L)
                zm = tot[g] < 1.0

                def qfix(qq, _):
                    sel = scorebuf[qbase + qq, ds] == -2.0
                    fm = jnp.where(sel, 1.0, 0.0)
                    row = mtile[qq, ds]
                    mtile[qq, ds] = jnp.where(zm, fm, row)
                    return 0

                lax.fori_loop(0, nrow, qfix, 0)
            pltpu.sync_copy(mtile.at[pl.ds(0, nrow), :], src)

        def mtile_fix(ti, _):
            qoff = pl.multiple_of(ti * 8, 8)
            fix_rows(qoff, 8,
                     out_hbm.at[pl.ds(qoff, 8), pl.ds(bcol, 128)])
            return 0

        lax.fori_loop(0, 12, mtile_fix, 0)
        fix_rows(96, 4, out_hbm.at[pl.ds(96, 4), pl.ds(bcol, 128)])

    pl.when(zany & (k == 0))(fallback)


def kernel(input_modal, detr_features, detr_logits):
    lt = jnp.transpose(detr_logits, (2, 1, 0))
    mask_t = _mask_kernel(lt)
    return detr_features, mask_t.T


# speedup vs baseline: 1.0055x; 1.0055x over previous
import functools

import jax
import jax.numpy as jnp
from jax import lax
from jax.experimental import pallas as pl
from jax.experimental.pallas import tpu as pltpu
from jax.experimental.pallas import tpu_sc as plsc

B = 1024
Q = 100
C = 92
K = 4
L = 16
CH = 46
BIG = 10**9

_mesh = plsc.VectorSubcoreMesh(core_axis_name="c", subcore_axis_name="s")


@functools.partial(
    pl.kernel,
    mesh=_mesh,
    compiler_params=pltpu.CompilerParams(needs_layout_passes=False),
    out_type=jax.ShapeDtypeStruct((Q, B), jnp.float32),
    scratch_types=[
        pltpu.VMEM((CH, 8, 128), jnp.float32),
        pltpu.VMEM((CH, 8, 128), jnp.float32),
        pltpu.VMEM((8, 128), jnp.float32),
        pltpu.VMEM((8, 128), jnp.float32),
        pltpu.VMEM((128,), jnp.float32),
        pltpu.VMEM((4, 128), jnp.float32),
        pltpu.VMEM((Q, 128), jnp.float32),
        pltpu.VMEM_SHARED((16, 128), jnp.float32),
        pltpu.SemaphoreType.DMA,
        pltpu.SemaphoreType.DMA,
    ],
)
def _mask_kernel(lt_hbm, out_hbm, bufa, bufb, mpart, mtile, accbuf, pbuf,
                 scorebuf, shacc, sema, semb):
    core = lax.axis_index("c")
    s = lax.axis_index("s")
    tj = core * 4 + s // 4
    k = s % 4
    bcol = tj * 128

    def qoff_of(u):
        return pl.multiple_of((k * 3 + u) * 8, 8)

    def half_src(u, h):
        return lt_hbm.at[pl.ds(h * CH, CH), pl.ds(qoff_of(u), 8),
                         pl.ds(bcol, 128)]

    def tail_src(h):
        return lt_hbm.at[pl.ds(h * CH, CH), pl.ds(96, 4), pl.ds(bcol, 128)]

    for g in range(8):
        accbuf[pl.ds(g * L, L)] = jnp.zeros((L,), jnp.float32)

    pltpu.async_copy(half_src(0, 0), bufa, sema)

    def unit(u, _):
        qoff = qoff_of(u)
        pltpu.make_async_copy(half_src(u, 0), bufa, sema).wait()
        pltpu.async_copy(half_src(u, 1), bufb, semb)

        def p1row(r, _):
            qq, g = r // 8, r % 8
            m = bufa[0, qq, pl.ds(g * L, L)]

            def cstep(i, mm):
                c0 = 1 + i * 9
                for d in range(9):
                    mm = jnp.maximum(mm, bufa[c0 + d, qq, pl.ds(g * L, L)])
                return mm

            mpart[qq, pl.ds(g * L, L)] = lax.fori_loop(0, 5, cstep, m)
            return 0

        lax.fori_loop(0, 64, p1row, 0)
        pltpu.make_async_copy(half_src(u, 1), bufb, semb).wait()

        @pl.when(u < 2)
        def _():
            pltpu.async_copy(half_src(u + 1, 0), bufa, sema)

        def p2row(r, _):
            qq, g = r // 8, r % 8
            m = mpart[qq, pl.ds(g * L, L)]

            def cstep(i, mm):
                c0 = i * 9
                for d in range(9):
                    mm = jnp.maximum(mm, bufb[c0 + d, qq, pl.ds(g * L, L)])
                return mm

            m = lax.fori_loop(0, 5, cstep, m)
            l91 = bufb[CH - 1, qq, pl.ds(g * L, L)]
            m1 = jnp.where(m >= l91, 1.0, 0.0)
            mtile[qq, pl.ds(g * L, L)] = m1
            accbuf[pl.ds(g * L, L)] = accbuf[pl.ds(g * L, L)] + m1
            return 0

        lax.fori_loop(0, 64, p2row, 0)
        pltpu.sync_copy(mtile,
                        out_hbm.at[pl.ds(qoff, 8), pl.ds(bcol, 128)])
        return 0

    lax.fori_loop(0, 3, unit, 0)

    @pl.when(k == 3)
    def _tail():
        pltpu.sync_copy(tail_src(0), bufa.at[:, pl.ds(0, 4), :])
        pltpu.sync_copy(tail_src(1), bufb.at[:, pl.ds(0, 4), :])

        def trow(r, _):
            qq, g = r // 8, r % 8
            m = bufa[0, qq, pl.ds(g * L, L)]

            def c1(i, mm):
                c0 = 1 + i * 9
                for d in range(9):
                    mm = jnp.maximum(mm, bufa[c0 + d, qq, pl.ds(g * L, L)])
                return mm

            m = lax.fori_loop(0, 5, c1, m)

            def c2(i, mm):
                c0 = i * 9
                for d in range(9):
                    mm = jnp.maximum(mm, bufb[c0 + d, qq, pl.ds(g * L, L)])
                return mm

            m = lax.fori_loop(0, 5, c2, m)
            l91 = bufb[CH - 1, qq, pl.ds(g * L, L)]
            m1 = jnp.where(m >= l91, 1.0, 0.0)
            mtile[qq, pl.ds(g * L, L)] = m1
            accbuf[pl.ds(g * L, L)] = accbuf[pl.ds(g * L, L)] + m1
            return 0

        lax.fori_loop(0, 32, trow, 0)
        pltpu.sync_copy(mtile.at[pl.ds(0, 4), :],
                        out_hbm.at[pl.ds(96, 4), pl.ds(bcol, 128)])

    pltpu.sync_copy(accbuf, shacc.at[s])
    plsc.subcore_barrier()
    pltpu.sync_copy(shacc.at[pl.ds((s // 4) * 4, 4)], pbuf)
    tot = [pbuf[0, pl.ds(g * L, L)] + pbuf[1, pl.ds(g * L, L)]
           + pbuf[2, pl.ds(g * L, L)] + pbuf[3, pl.ds(g * L, L)]
           for g in range(8)]
    zvec = sum(jnp.where(tot[g] < 1.0, 1.0, 0.0) for g in range(8))
    zany = jnp.max(zvec) > 0.0

    def fallback():
        def srows(bufx, bufy, qbase, nrow):
            def srow(r, _):
                qq, g = r // 8, r % 8
                ds = pl.ds(g * L, L)
                m = bufx[0, qq, ds]

                def c1(cc, mm):
                    return jnp.maximum(mm, bufx[cc, qq, ds])

                m = lax.fori_loop(1, CH, c1, m)

                def c2(cc, mm):
                    return jnp.maximum(mm, bufy[cc, qq, ds])

                m = lax.fori_loop(0, CH - 1, c2, m)

                def e1(cc, ss):
                    return ss + jnp.exp(bufx[cc, qq, ds] - m)

                ssum = lax.fori_loop(0, CH, e1,
                                     jnp.zeros((L,), jnp.float32))

                def e2(cc, ss):
                    return ss + jnp.exp(bufy[cc, qq, ds] - m)

                ssum = lax.fori_loop(0, CH - 1, e2, ssum)
                scorebuf[qbase + qq, ds] = 1.0 / ssum
                return 0

            lax.fori_loop(0, nrow * 8, srow, 0)

        def stile(ti, _):
            qoff = pl.multiple_of(ti * 8, 8)
            pltpu.sync_copy(lt_hbm.at[pl.ds(0, CH), pl.ds(qoff, 8),
                                      pl.ds(bcol, 128)], bufa)
            pltpu.sync_copy(lt_hbm.at[pl.ds(CH, CH), pl.ds(qoff, 8),
                                      pl.ds(bcol, 128)], bufb)
            srows(bufa, bufb, qoff, 8)
            return 0

        lax.fori_loop(0, 12, stile, 0)
        pltpu.sync_copy(tail_src(0), bufa.at[:, pl.ds(0, 4), :])
        pltpu.sync_copy(tail_src(1), bufb.at[:, pl.ds(0, 4), :])
        srows(bufa, bufb, 96, 4)

        def kround(_, __):
            for g in range(8):
                ds = pl.ds(g * L, L)
                mx = scorebuf[0, ds]

                def qmax(q, mm):
                    return jnp.maximum(mm, scorebuf[q, ds])

                mx = lax.fori_loop(1, Q, qmax, mx)

                def qidx(q, bi):
                    return jnp.minimum(
                        bi, jnp.where(scorebuf[q, ds] == mx, q, BIG))

                bidx = lax.fori_loop(0, Q, qidx,
                                     jnp.full((L,), BIG, jnp.int32))

                def qkill(q, _):
                    row = scorebuf[q, ds]
                    scorebuf[q, ds] = jnp.where(bidx == q, -2.0, row)
                    return 0

                lax.fori_loop(0, Q, qkill, 0)
            return 0

        lax.fori_loop(0, K, kround, 0)

        def fix_rows(qbase, nrow, src):
            pltpu.sync_copy(src, mtile.at[pl.ds(0, nrow), :])
            for g in range(8):
                ds = pl.ds(g * L, L)
                zm = tot[g] < 1.0

                def qfix(qq, _):
                    sel = scorebuf[qbase + qq, ds] == -2.0
                    fm = jnp.where(sel, 1.0, 0.0)
                    row = mtile[qq, ds]
                    mtile[qq, ds] = jnp.where(zm, fm, row)
                    return 0

                lax.fori_loop(0, nrow, qfix, 0)
            pltpu.sync_copy(mtile.at[pl.ds(0, nrow), :], src)

        def mtile_fix(ti, _):
            qoff = pl.multiple_of(ti * 8, 8)
            fix_rows(qoff, 8,
                     out_hbm.at[pl.ds(qoff, 8), pl.ds(bcol, 128)])
            return 0

        lax.fori_loop(0, 12, mtile_fix, 0)
        fix_rows(96, 4, out_hbm.at[pl.ds(96, 4), pl.ds(bcol, 128)])

    pl.when(zany & (k == 0))(fallback)


def kernel(input_modal, detr_features, detr_logits):
    lt = jnp.transpose(detr_logits, (2, 1, 0))
    mask_t = _mask_kernel(lt)
    return detr_features, mask_t.T
